# baseline (speedup 1.0000x reference)
import jax
import jax.numpy as jnp
from jax import lax
from jax.experimental import pallas as pl
from jax.experimental.pallas import tpu as pltpu
from jax.experimental.pallas import tpu_sc as plsc

VOCAB = 1000000
DIM = 64
PAD = 0

_NC = 2
_NS = 16
_L = 16
_NW = _NC * _NS

_B = 8192
_BPW = _B // _NW
_IDX_MINOR = 128
_ROWS_PER_W = _BPW // _IDX_MINOR
_PACK = 4
_PW = 128
_QW = _PW // _PACK


def _sc_body(idx_hbm, idxq_hbm, table_hbm, out_hbm, idx_v, idxq_v, rows_v, sem):
    wid = lax.axis_index("s") * _NC + lax.axis_index("c")
    base = wid * _BPW

    pltpu.sync_copy(idx_hbm.at[pl.ds(_ROWS_PER_W * wid, _ROWS_PER_W)], idx_v)
    pltpu.sync_copy(idxq_hbm.at[pl.ds(_ROWS_PER_W * wid, _ROWS_PER_W)], idxq_v)

    copies = []
    for j in range(_ROWS_PER_W):
        copies.append(
            pltpu.async_copy(
                table_hbm.at[idxq_v.at[j]],
                rows_v.at[pl.ds(j * _IDX_MINOR, _IDX_MINOR), :],
                sem,
            )
        )
    for c in copies:
        c.wait()

    ones = jnp.ones((_L,), jnp.int32)
    zeros = jnp.zeros((_L,), jnp.int32)
    acc = zeros
    for j in range(_ROWS_PER_W):
        for t in range(_IDX_MINOR // _L):
            iv = idx_v[j, pl.ds(t * _L, _L)]
            acc = acc | jnp.where(iv == PAD, ones, zeros)
    has_pad = jnp.max(acc)

    @pl.when(has_pad > 0)
    def _zero_pad_rows():
        dnums = lax.GatherDimensionNumbers(
            offset_dims=(), collapsed_slice_dims=(0,), start_index_map=(0,))
        for j in range(_ROWS_PER_W):
            for t in range(_IDX_MINOR // _L):
                iv = idx_v[j, pl.ds(t * _L, _L)]
                scale = jnp.where(iv == PAD, zeros, ones)
                for r in range(_L):
                    row = j * _IDX_MINOR + t * _L + r
                    bidx = jnp.full((_L, 1), r, jnp.int32)
                    bvec = lax.gather(
                        scale, bidx, dnums, (1,),
                        mode=lax.GatherScatterMode.PROMISE_IN_BOUNDS)
                    for cchunk in range(_QW // _L):
                        sl = pl.ds(cchunk * _L, _L)
                        rows_v[row, sl] = rows_v[row, sl] * bvec

    pltpu.sync_copy(rows_v, out_hbm.at[pl.ds(base, _BPW)])


def _lookup(idx2d, idxq2d, packed_tbl):
    mesh = plsc.VectorSubcoreMesh(core_axis_name="c", subcore_axis_name="s")
    return pl.kernel(
        _sc_body,
        out_type=jax.ShapeDtypeStruct((_B, _PW), jnp.int32),
        mesh=mesh,
        compiler_params=pltpu.CompilerParams(needs_layout_passes=False),
        scratch_types=[
            pltpu.VMEM((_ROWS_PER_W, _IDX_MINOR), jnp.int32),
            pltpu.VMEM((_ROWS_PER_W, _IDX_MINOR), jnp.int32),
            pltpu.VMEM((_BPW, _PW), jnp.int32),
            pltpu.SemaphoreType.DMA,
        ],
    )(idx2d, idxq2d, packed_tbl)


def kernel(src_input, word_lut):
    seq, batch, _ = src_input.shape
    idx = src_input[:, :, 0].reshape(_B)
    tbl16 = word_lut.astype(jnp.bfloat16)
    packed = lax.bitcast_convert_type(
        tbl16.reshape(VOCAB // _PACK, _PW, 2), jnp.int32)
    idx2d = idx.reshape(_B // _IDX_MINOR, _IDX_MINOR)
    idxq2d = (idx >> 2).reshape(_B // _IDX_MINOR, _IDX_MINOR)
    wide = _lookup(idx2d, idxq2d, packed)
    wide16 = lax.bitcast_convert_type(wide, jnp.bfloat16).reshape(_B, 2 * _PW)
    q = (idx & (_PACK - 1))[:, None]
    emb16 = jnp.where(
        q < 2,
        jnp.where(q == 0, wide16[:, 0:DIM], wide16[:, DIM:2 * DIM]),
        jnp.where(q == 2, wide16[:, 2 * DIM:3 * DIM], wide16[:, 3 * DIM:]),
    )
    return emb16.astype(jnp.float32).reshape(seq, batch, DIM)


# speedup vs baseline: 35.2444x; 35.2444x over previous
import jax
import jax.numpy as jnp
from jax import lax
from jax.experimental import pallas as pl
from jax.experimental.pallas import tpu as pltpu
from jax.experimental.pallas import tpu_sc as plsc

VOCAB = 1000000
DIM = 64
PAD = 0

_NC = 2
_NS = 16
_L = 16
_NW = _NC * _NS

_B = 8192
_BPW = _B // _NW
_IDX_MINOR = 128
_ROWS_PER_W = _BPW // _IDX_MINOR
_PACK = 2
_PW = 128
_QW = _PW // _PACK


def _sc_body(idx_hbm, idxq_hbm, table_hbm, out_hbm, idx_v, idxq_v, rows_v, sem):
    wid = lax.axis_index("s") * _NC + lax.axis_index("c")
    base = wid * _BPW

    pltpu.sync_copy(idx_hbm.at[pl.ds(_ROWS_PER_W * wid, _ROWS_PER_W)], idx_v)
    pltpu.sync_copy(idxq_hbm.at[pl.ds(_ROWS_PER_W * wid, _ROWS_PER_W)], idxq_v)

    copies = []
    for j in range(_ROWS_PER_W):
        copies.append(
            pltpu.async_copy(
                table_hbm.at[idxq_v.at[j]],
                rows_v.at[pl.ds(j * _IDX_MINOR, _IDX_MINOR), :],
                sem,
            )
        )
    for c in copies:
        c.wait()

    ones = jnp.ones((_L,), jnp.int32)
    zeros = jnp.zeros((_L,), jnp.int32)
    acc = zeros
    for j in range(_ROWS_PER_W):
        for t in range(_IDX_MINOR // _L):
            iv = idx_v[j, pl.ds(t * _L, _L)]
            acc = acc | jnp.where(iv == PAD, ones, zeros)
    has_pad = jnp.max(acc)

    @pl.when(has_pad > 0)
    def _zero_pad_rows():
        dnums = lax.GatherDimensionNumbers(
            offset_dims=(), collapsed_slice_dims=(0,), start_index_map=(0,))
        for j in range(_ROWS_PER_W):
            for t in range(_IDX_MINOR // _L):
                iv = idx_v[j, pl.ds(t * _L, _L)]
                scale = jnp.where(iv == PAD, zeros, ones)
                for r in range(_L):
                    row = j * _IDX_MINOR + t * _L + r
                    bidx = jnp.full((_L, 1), r, jnp.int32)
                    bvec = lax.gather(
                        scale, bidx, dnums, (1,),
                        mode=lax.GatherScatterMode.PROMISE_IN_BOUNDS)
                    for cchunk in range(_QW // _L):
                        sl = pl.ds(cchunk * _L, _L)
                        rows_v[row, sl] = rows_v[row, sl] * bvec

    pltpu.sync_copy(rows_v, out_hbm.at[pl.ds(base, _BPW)])


def _lookup(idx2d, idxq2d, packed_tbl):
    mesh = plsc.VectorSubcoreMesh(core_axis_name="c", subcore_axis_name="s")
    return pl.kernel(
        _sc_body,
        out_type=jax.ShapeDtypeStruct((_B, _PW), jnp.float32),
        mesh=mesh,
        compiler_params=pltpu.CompilerParams(needs_layout_passes=False),
        scratch_types=[
            pltpu.VMEM((_ROWS_PER_W, _IDX_MINOR), jnp.int32),
            pltpu.VMEM((_ROWS_PER_W, _IDX_MINOR), jnp.int32),
            pltpu.VMEM((_BPW, _PW), jnp.float32),
            pltpu.SemaphoreType.DMA,
        ],
    )(idx2d, idxq2d, packed_tbl)


def kernel(src_input, word_lut):
    seq, batch, _ = src_input.shape
    idx = src_input[:, :, 0].reshape(_B)
    packed = word_lut.reshape(VOCAB // _PACK, _PW)
    idx2d = idx.reshape(_B // _IDX_MINOR, _IDX_MINOR)
    idxq2d = (idx >> 1).reshape(_B // _IDX_MINOR, _IDX_MINOR)
    wide = _lookup(idx2d, idxq2d, packed)
    q = (idx & (_PACK - 1))[:, None]
    emb = jnp.where(q == 0, wide[:, 0:DIM], wide[:, DIM:])
    return emb.reshape(seq, batch, DIM)


# speedup vs baseline: 172.2687x; 4.8878x over previous
import jax
import jax.numpy as jnp
from jax import lax
from jax.experimental import pallas as pl
from jax.experimental.pallas import tpu as pltpu
from jax.experimental.pallas import tpu_sc as plsc

VOCAB = 1000000
DIM = 64
PAD = 0

_NC = 2
_NS = 16
_L = 16
_NW = _NC * _NS

_B = 8192
_BPW = _B // _NW
_NB = 8
_TC = 128
_LASTBASE = (VOCAB // _TC - 1) * _TC
_NG = _BPW // _L


def _sc_body(idx_hbm, lutT_hbm, out_hbm, idx_v, rows_v,
             b0, b1, b2, b3, b4, b5, b6, b7,
             s0, s1, s2, s3, s4, s5, s6, s7):
    sems = (s0, s1, s2, s3, s4, s5, s6, s7)
    wid = lax.axis_index("s") * _NC + lax.axis_index("c")
    base = wid * _BPW

    for j in range(2):
        pltpu.sync_copy(idx_hbm.at[2 * wid + j],
                        idx_v.at[pl.ds(j * 128, 128)])

    bufs = (b0, b1, b2, b3, b4, b5, b6, b7)
    lastb = jnp.full((_L,), _LASTBASE, jnp.int32)
    onesf = jnp.ones((_L,), jnp.float32)
    zerosf = jnp.zeros((_L,), jnp.float32)
    dnums = lax.GatherDimensionNumbers(
        offset_dims=(), collapsed_slice_dims=(0,), start_index_map=(0,))

    def group_vecs(g):
        iv = idx_v[pl.ds(g * _L, _L)]
        bcol = jnp.minimum((iv >> 7) << 7, lastb)
        return iv, bcol

    def fire(k, bc_s):
        return pltpu.async_copy(
            lutT_hbm.at[:, pl.ds(pl.multiple_of(bc_s, _TC), _TC)],
            bufs[k % _NB],
            sems[k % _NB],
        )

    handles = [None] * _BPW

    iv0, bc0 = group_vecs(0)
    for r in range(_NB):
        handles[r] = fire(r, bc0[r])

    for g in range(_NG):
        iv_g, bc_g = group_vecs(g)
        colrel = iv_g - bc_g
        scale = jnp.where(iv_g == PAD, zerosf, onesf)
        if g + 1 < _NG:
            _, bc_n = group_vecs(g + 1)
        for r in range(_L):
            k = g * _L + r
            handles[k].wait()
            bidx = jnp.full((_L, 1), r, jnp.int32)
            col_b = lax.gather(colrel, bidx, dnums, (1,),
                               mode=lax.GatherScatterMode.PROMISE_IN_BOUNDS)
            sc_b = lax.gather(scale, bidx, dnums, (1,),
                              mode=lax.GatherScatterMode.PROMISE_IN_BOUNDS)
            buf = bufs[k % _NB]
            for m in range(DIM // _L):
                dvec = lax.iota(jnp.int32, _L) + (m * _L)
                val = plsc.load_gather(buf, [dvec, col_b])
                rows_v[k, pl.ds(m * _L, _L)] = val * sc_b
            kf = k + _NB
            if kf < _BPW:
                bc_s = bc_g[r + _NB] if r + _NB < _L else bc_n[r + _NB - _L]
                handles[kf] = fire(kf, bc_s)

    pltpu.sync_copy(rows_v, out_hbm.at[pl.ds(base, _BPW)])


def _lookup(idx2d, lutT):
    mesh = plsc.VectorSubcoreMesh(core_axis_name="c", subcore_axis_name="s")
    return pl.kernel(
        _sc_body,
        out_type=jax.ShapeDtypeStruct((_B, DIM), jnp.float32),
        mesh=mesh,
        compiler_params=pltpu.CompilerParams(needs_layout_passes=False),
        scratch_types=[
            pltpu.VMEM((_BPW,), jnp.int32),
            pltpu.VMEM((_BPW, DIM), jnp.float32),
        ] + [pltpu.VMEM((DIM, _TC), jnp.float32)] * _NB
          + [pltpu.SemaphoreType.DMA] * _NB,
    )(idx2d, lutT)


def kernel(src_input, word_lut):
    seq, batch, _ = src_input.shape
    idx2d = src_input[:, :, 0].reshape(_B // 128, 128)
    out = _lookup(idx2d, word_lut.T)
    return out.reshape(seq, batch, DIM)


# speedup vs baseline: 174.8675x; 1.0151x over previous
import jax
import jax.numpy as jnp
from jax import lax
from jax.experimental import pallas as pl
from jax.experimental.pallas import tpu as pltpu
from jax.experimental.pallas import tpu_sc as plsc

VOCAB = 1000000
DIM = 64
PAD = 0

_NC = 2
_NS = 16
_L = 16
_NW = _NC * _NS

_B = 8192
_BPW = _B // _NW
_NB = 10
_TC = 128
_LASTBASE = (VOCAB // _TC - 1) * _TC
_NG = _BPW // _L


def _sc_body(idx_hbm, lutT_hbm, out_hbm, idx_v, rows_v, *rest):
    bufs = rest[:_NB]
    sems = rest[_NB:]
    wid = lax.axis_index("s") * _NC + lax.axis_index("c")
    base = wid * _BPW

    for j in range(2):
        pltpu.sync_copy(idx_hbm.at[2 * wid + j],
                        idx_v.at[pl.ds(j * 128, 128)])

    onesf = jnp.ones((_L,), jnp.float32)
    zerosf = jnp.zeros((_L,), jnp.float32)
    dnums = lax.GatherDimensionNumbers(
        offset_dims=(), collapsed_slice_dims=(0,), start_index_map=(0,))

    def group_vecs(g):
        iv = idx_v[pl.ds(g * _L, _L)]
        return iv, (iv >> 7) << 7

    def fire(k, bc_s):
        return pltpu.async_copy(
            lutT_hbm.at[:, pl.ds(pl.multiple_of(bc_s, _TC), _TC)],
            bufs[k % _NB],
            sems[k % _NB],
        )

    dvecs = [lax.iota(jnp.int32, _L) + (m * _L) for m in range(DIM // _L)]
    handles = [None] * _BPW

    iv0, bc0 = group_vecs(0)
    for r in range(_NB):
        handles[r] = fire(r, bc0[r])

    for g in range(_NG):
        iv_g, bc_g = group_vecs(g)
        colrel = iv_g - bc_g
        scale = jnp.where(iv_g == PAD, zerosf, onesf)
        if g + 1 < _NG:
            _, bc_n = group_vecs(g + 1)
        for r in range(_L):
            k = g * _L + r
            handles[k].wait()
            bidx = jnp.full((_L, 1), r, jnp.int32)
            col_b = lax.gather(colrel, bidx, dnums, (1,),
                               mode=lax.GatherScatterMode.PROMISE_IN_BOUNDS)
            sc_b = lax.gather(scale, bidx, dnums, (1,),
                              mode=lax.GatherScatterMode.PROMISE_IN_BOUNDS)
            buf = bufs[k % _NB]
            for m in range(DIM // _L):
                val = plsc.load_gather(buf, [dvecs[m], col_b])
                rows_v[k, pl.ds(m * _L, _L)] = val * sc_b
            kf = k + _NB
            if kf < _BPW:
                bc_s = bc_g[r + _NB] if r + _NB < _L else bc_n[r + _NB - _L]
                handles[kf] = fire(kf, bc_s)

    pltpu.sync_copy(rows_v, out_hbm.at[pl.ds(base, _BPW)])


def _lookup(idx2d, lutT):
    mesh = plsc.VectorSubcoreMesh(core_axis_name="c", subcore_axis_name="s")
    return pl.kernel(
        _sc_body,
        out_type=jax.ShapeDtypeStruct((_B, DIM), jnp.float32),
        mesh=mesh,
        compiler_params=pltpu.CompilerParams(needs_layout_passes=False),
        scratch_types=[
            pltpu.VMEM((_BPW,), jnp.int32),
            pltpu.VMEM((_BPW, DIM), jnp.float32),
        ] + [pltpu.VMEM((DIM, _TC), jnp.float32)] * _NB
          + [pltpu.SemaphoreType.DMA] * _NB,
    )(idx2d, lutT)


def kernel(src_input, word_lut):
    seq, batch, _ = src_input.shape
    idx2d = src_input[:, :, 0].reshape(_B // 128, 128)
    out = _lookup(idx2d, word_lut.T)
    return out.reshape(seq, batch, DIM)
